# baseline (speedup 1.0000x reference)
import functools

import jax
import jax.numpy as jnp
from jax import lax
from jax.experimental import pallas as pl
from jax.experimental.pallas import tpu as pltpu
from jax.experimental.pallas import tpu_sc as plsc

_NS1 = 17
_B, _L, _D = 4096, 200, 64
_BL = _B * _L
_M = _BL * _D // 128

_NC, _NSUB = 2, 16
_NW = _NC * _NSUB
_CHUNK = _BL // _NW

_KS1 = 42
_KS2 = 42 ^ 0x1BD11BDA
_R1 = (13, 15, 26, 6)
_R2 = (17, 29, 16, 24)

_Q = (-3.730023951734319e-09, -1.747619631184354e-07, -1.1426428921245524e-06,
      6.983217098337471e-05, 0.0015873134248983374, 0.008801878692352457,
      -0.2266867857871575, 1.2534667757445634)

_LO = -0.9999999403953552

_RB = 4096


def _u32(x):
    return jnp.uint32(x)


def _round(x0, x1, r):
    x0 = x0 + x1
    x1 = (lax.shift_left(x1, _u32(r)) | lax.shift_right_logical(x1, _u32(32 - r))) ^ x0
    return x0, x1


def _threefry_0_42(cnt):
    x1i = cnt + _u32(_KS1)
    x0 = x1i
    x1 = (lax.shift_left(x1i, _u32(13)) | lax.shift_right_logical(x1i, _u32(19))) ^ x0
    for r in _R1[1:]:
        x0, x1 = _round(x0, x1, r)
    x0 = x0 + _u32(_KS1)
    x1 = x1 + _u32((_KS2 + 1) & 0xFFFFFFFF)
    for r in _R2:
        x0, x1 = _round(x0, x1, r)
    x0 = x0 + _u32(_KS2)
    x1 = x1 + _u32(2)
    for r in _R1:
        x0, x1 = _round(x0, x1, r)
    x1 = x1 + _u32((_KS1 + 3) & 0xFFFFFFFF)
    for r in _R2:
        x0, x1 = _round(x0, x1, r)
    x0 = x0 + _u32(_KS1)
    x1 = x1 + _u32((_KS2 + 4) & 0xFFFFFFFF)
    for r in _R1:
        x0, x1 = _round(x0, x1, r)
    x0 = x0 + _u32(_KS2)
    x1 = x1 + _u32(5)
    return x0 ^ x1


def _tc_noise_body(sel_ref, out_ref, *, rb):
    i = pl.program_id(0)
    base = i * (rb * 128)
    r = lax.broadcasted_iota(jnp.int32, (rb, 128), 0)
    c = lax.broadcasted_iota(jnp.int32, (rb, 128), 1)
    cnt = (base + r * 64 + (c & 63) + (c >> 6) * (rb * 64)).astype(jnp.uint32)
    bits = _threefry_0_42(cnt)
    fb = lax.shift_right_logical(bits, _u32(9)) | _u32(0x3F800000)
    f = lax.bitcast_convert_type(fb, jnp.float32) - jnp.float32(1.0)
    lo = jnp.float32(_LO)
    u = f * (jnp.float32(1.0) - lo) + lo
    t = jnp.log2((jnp.float32(1.0) - u) * (jnp.float32(1.0) + u))
    q = jnp.float32(_Q[0])
    for cc in _Q[1:]:
        q = q * t + jnp.float32(cc)
    sig = jnp.where(c < 64, sel_ref[:, 0:1], sel_ref[:, 1:2])
    y = q * u * sig
    out_ref[pl.ds(0, rb), :] = y[:, 0:64]
    out_ref[pl.ds(rb, rb), :] = y[:, 64:128]


def _sc_gather_body(idx_hbm, sig_hbm, out_hbm, i_v, o_v, a_v, sem):
    wid = lax.axis_index("s") * _NC + lax.axis_index("c")
    base = wid * _CHUNK
    pltpu.sync_copy(idx_hbm.at[pl.ds(base, _CHUNK)], i_v)
    pltpu.async_copy(sig_hbm.at[i_v], o_v, sem).wait()
    step = lax.iota(jnp.int32, 16) * 128

    def body(j, carry):
        s = pl.ds(j * 16, 16)
        k = base + j * 16
        addr = (k // (2 * _RB)) * (_RB * 128) + (k % _RB) * 128 + (k // _RB) % 2
        a_v[s] = addr + step
        return carry

    lax.fori_loop(0, _CHUNK // 16, body, 0)
    pltpu.async_copy(o_v, out_hbm.at[a_v], sem).wait()


def _make_sc_gather():
    return functools.partial(
        pl.kernel,
        out_type=jax.ShapeDtypeStruct((_BL * 64,), jnp.float32),
        mesh=plsc.VectorSubcoreMesh(
            core_axis_name="c", subcore_axis_name="s",
            num_cores=_NC, num_subcores=_NSUB,
        ),
        scratch_types=[
            pltpu.VMEM((_CHUNK,), jnp.int32),
            pltpu.VMEM((_CHUNK,), jnp.float32),
            pltpu.VMEM((_CHUNK,), jnp.int32),
            pltpu.SemaphoreType.DMA,
        ],
    )(_sc_gather_body)


def kernel(concepts, indices, embeddings, sigmas):
    del embeddings
    flat_idx = (concepts * _NS1 + indices).reshape(_BL)
    selected = _make_sc_gather()(flat_idx, sigmas.reshape(-1))

    rb = _RB
    out = pl.pallas_call(
        functools.partial(_tc_noise_body, rb=rb),
        grid=(_M // rb,),
        in_specs=[pl.BlockSpec((rb, 128), lambda i: (i, 0))],
        out_specs=pl.BlockSpec((2 * rb, 64), lambda i: (i, 0)),
        out_shape=jax.ShapeDtypeStruct((_BL, 64), jnp.float32),
        compiler_params=pltpu.CompilerParams(
            dimension_semantics=("parallel",),
        ),
    )(selected.reshape(_M, 128))
    return out.reshape(_B, _L, _D)


# speedup vs baseline: 2.1310x; 2.1310x over previous
import functools

import jax
import jax.numpy as jnp
from jax import lax
from jax.experimental import pallas as pl
from jax.experimental.pallas import tpu as pltpu
from jax.experimental.pallas import tpu_sc as plsc

_NS1 = 17
_B, _L, _D = 4096, 200, 64
_BL = _B * _L
_M = _BL * _D // 128

_NC, _NSUB = 2, 16
_NW = _NC * _NSUB
_CHUNK = _BL // _NW

_KS1 = 42
_KS2 = 42 ^ 0x1BD11BDA
_R1 = (13, 15, 26, 6)
_R2 = (17, 29, 16, 24)

_Q = (-3.730023951734319e-09, -1.747619631184354e-07, -1.1426428921245524e-06,
      6.983217098337471e-05, 0.0015873134248983374, 0.008801878692352457,
      -0.2266867857871575, 1.2534667757445634)

_LO = -0.9999999403953552

_RB = 4096


def _u32(x):
    return jnp.uint32(x)


def _round(x0, x1, r):
    x0 = x0 + x1
    x1 = (lax.shift_left(x1, _u32(r)) | lax.shift_right_logical(x1, _u32(32 - r))) ^ x0
    return x0, x1


def _threefry_0_42(cnt):
    x1i = cnt + _u32(_KS1)
    x0 = x1i
    x1 = (lax.shift_left(x1i, _u32(13)) | lax.shift_right_logical(x1i, _u32(19))) ^ x0
    for r in _R1[1:]:
        x0, x1 = _round(x0, x1, r)
    x0 = x0 + _u32(_KS1)
    x1 = x1 + _u32((_KS2 + 1) & 0xFFFFFFFF)
    for r in _R2:
        x0, x1 = _round(x0, x1, r)
    x0 = x0 + _u32(_KS2)
    x1 = x1 + _u32(2)
    for r in _R1:
        x0, x1 = _round(x0, x1, r)
    x1 = x1 + _u32((_KS1 + 3) & 0xFFFFFFFF)
    for r in _R2:
        x0, x1 = _round(x0, x1, r)
    x0 = x0 + _u32(_KS1)
    x1 = x1 + _u32((_KS2 + 4) & 0xFFFFFFFF)
    for r in _R1:
        x0, x1 = _round(x0, x1, r)
    x0 = x0 + _u32(_KS2)
    x1 = x1 + _u32(5)
    return x0 ^ x1


def _tc_noise_body(sel_ref, out_ref, *, bk):
    i = pl.program_id(0)
    base = i * bk
    r = lax.broadcasted_iota(jnp.int32, (bk, 4096), 0)
    b = lax.broadcasted_iota(jnp.int32, (bk, 4096), 1)
    cnt = (b * 12800 + (base + r)).astype(jnp.uint32)
    bits = _threefry_0_42(cnt)
    fb = lax.shift_right_logical(bits, _u32(9)) | _u32(0x3F800000)
    f = lax.bitcast_convert_type(fb, jnp.float32) - jnp.float32(1.0)
    lo = jnp.float32(_LO)
    u = f * (jnp.float32(1.0) - lo) + lo
    t = jnp.log2((jnp.float32(1.0) - u) * (jnp.float32(1.0) + u))
    q = jnp.float32(_Q[0])
    for cc in _Q[1:]:
        q = q * t + jnp.float32(cc)
    sig = jnp.where(r < 64, sel_ref[0, 0:1, :], sel_ref[0, 1:2, :])
    out_ref[...] = q * u * sig


def _sc_gather_body(idx_hbm, sig_hbm, out_hbm, i_v, o_v, sem):
    wid = lax.axis_index("s") * _NC + lax.axis_index("c")
    base = wid * _CHUNK
    pltpu.sync_copy(idx_hbm.at[pl.ds(base, _CHUNK)], i_v)
    pltpu.async_copy(sig_hbm.at[i_v], o_v, sem).wait()
    pltpu.sync_copy(o_v, out_hbm.at[pl.ds(base, _CHUNK)])


def _make_sc_gather():
    return functools.partial(
        pl.kernel,
        out_type=jax.ShapeDtypeStruct((_BL,), jnp.float32),
        mesh=plsc.VectorSubcoreMesh(
            core_axis_name="c", subcore_axis_name="s",
            num_cores=_NC, num_subcores=_NSUB,
        ),
        scratch_types=[
            pltpu.VMEM((_CHUNK,), jnp.int32),
            pltpu.VMEM((_CHUNK,), jnp.float32),
            pltpu.SemaphoreType.DMA,
        ],
    )(_sc_gather_body)


def kernel(concepts, indices, embeddings, sigmas):
    del embeddings
    flat_idx = (indices.T * 100000 + concepts.T).reshape(_BL)
    selected = _make_sc_gather()(flat_idx, sigmas.T.reshape(-1))

    bk = 128
    out = pl.pallas_call(
        functools.partial(_tc_noise_body, bk=bk),
        grid=(_L * _D // bk,),
        in_specs=[pl.BlockSpec((1, 2, 4096), lambda i: (i, 0, 0))],
        out_specs=pl.BlockSpec((bk, 4096), lambda i: (i, 0)),
        out_shape=jax.ShapeDtypeStruct((_L * _D, _B), jnp.float32),
        compiler_params=pltpu.CompilerParams(
            dimension_semantics=("parallel",),
        ),
    )(selected.reshape(_L // 2, 2, _B))
    return out.reshape(_L, _D, _B).transpose(2, 0, 1)


# speedup vs baseline: 2.1889x; 1.0272x over previous
import functools

import jax
import jax.numpy as jnp
from jax import lax
from jax.experimental import pallas as pl
from jax.experimental.pallas import tpu as pltpu
from jax.experimental.pallas import tpu_sc as plsc

_NS1 = 17
_B, _L, _D = 4096, 200, 64
_BL = _B * _L
_M = _BL * _D // 128

_NC, _NSUB = 2, 16
_NW = _NC * _NSUB
_CHUNK = _BL // _NW

_KS1 = 42
_KS2 = 42 ^ 0x1BD11BDA
_R1 = (13, 15, 26, 6)
_R2 = (17, 29, 16, 24)

_Q = (1.717153855847334e-06, 8.895356981760094e-05, 0.0016275906203618575,
      0.00874971574873464, -0.2268891676002655, 1.253377167895753)

_LO = -0.9999999403953552

_RB = 4096


def _u32(x):
    return jnp.uint32(x)


def _round(x0, x1, r):
    x0 = x0 + x1
    x1 = (lax.shift_left(x1, _u32(r)) | lax.shift_right_logical(x1, _u32(32 - r))) ^ x0
    return x0, x1


def _threefry_0_42(cnt):
    x1i = cnt + _u32(_KS1)
    x0 = x1i
    x1 = (lax.shift_left(x1i, _u32(13)) | lax.shift_right_logical(x1i, _u32(19))) ^ x0
    for r in _R1[1:]:
        x0, x1 = _round(x0, x1, r)
    x0 = x0 + _u32(_KS1)
    x1 = x1 + _u32((_KS2 + 1) & 0xFFFFFFFF)
    for r in _R2:
        x0, x1 = _round(x0, x1, r)
    x0 = x0 + _u32(_KS2)
    x1 = x1 + _u32(2)
    for r in _R1:
        x0, x1 = _round(x0, x1, r)
    x1 = x1 + _u32((_KS1 + 3) & 0xFFFFFFFF)
    for r in _R2:
        x0, x1 = _round(x0, x1, r)
    x0 = x0 + _u32(_KS1)
    x1 = x1 + _u32((_KS2 + 4) & 0xFFFFFFFF)
    for r in _R1:
        x0, x1 = _round(x0, x1, r)
    x0 = x0 + _u32(_KS2)
    x1 = x1 + _u32(5)
    return x0 ^ x1


def _tc_noise_body(sel_ref, out_ref, *, bk):
    i = pl.program_id(0)
    base = i * bk
    r = lax.broadcasted_iota(jnp.int32, (bk, 4096), 0)
    b = lax.broadcasted_iota(jnp.int32, (bk, 4096), 1)
    cnt = (b * 12800 + (base + r)).astype(jnp.uint32)
    bits = _threefry_0_42(cnt)
    fb = lax.shift_right_logical(bits, _u32(9)) | _u32(0x3F800000)
    f = lax.bitcast_convert_type(fb, jnp.float32) - jnp.float32(1.0)
    lo = jnp.float32(_LO)
    u = f * (jnp.float32(1.0) - lo) + lo
    t = jnp.log2((jnp.float32(1.0) - u) * (jnp.float32(1.0) + u))
    q = jnp.float32(_Q[0])
    for cc in _Q[1:]:
        q = q * t + jnp.float32(cc)
    sig = jnp.where(r < 64, sel_ref[0, 0:1, :], sel_ref[0, 1:2, :])
    out_ref[...] = q * u * sig


def _sc_gather_body(idx_hbm, sig_hbm, out_hbm, i_v, o_v, sem):
    wid = lax.axis_index("s") * _NC + lax.axis_index("c")
    base = wid * _CHUNK
    pltpu.sync_copy(idx_hbm.at[pl.ds(base, _CHUNK)], i_v)
    pltpu.async_copy(sig_hbm.at[i_v], o_v, sem).wait()
    pltpu.sync_copy(o_v, out_hbm.at[pl.ds(base, _CHUNK)])


def _make_sc_gather():
    return functools.partial(
        pl.kernel,
        out_type=jax.ShapeDtypeStruct((_BL,), jnp.float32),
        mesh=plsc.VectorSubcoreMesh(
            core_axis_name="c", subcore_axis_name="s",
            num_cores=_NC, num_subcores=_NSUB,
        ),
        scratch_types=[
            pltpu.VMEM((_CHUNK,), jnp.int32),
            pltpu.VMEM((_CHUNK,), jnp.float32),
            pltpu.SemaphoreType.DMA,
        ],
    )(_sc_gather_body)


def kernel(concepts, indices, embeddings, sigmas):
    del embeddings
    flat_idx = (indices.T * 100000 + concepts.T).reshape(_BL)
    selected = _make_sc_gather()(flat_idx, sigmas.T.reshape(-1))

    bk = 128
    out = pl.pallas_call(
        functools.partial(_tc_noise_body, bk=bk),
        grid=(_L * _D // bk,),
        in_specs=[pl.BlockSpec((1, 2, 4096), lambda i: (i, 0, 0))],
        out_specs=pl.BlockSpec((bk, 4096), lambda i: (i, 0)),
        out_shape=jax.ShapeDtypeStruct((_L * _D, _B), jnp.float32),
        compiler_params=pltpu.CompilerParams(
            dimension_semantics=("parallel",),
        ),
    )(selected.reshape(_L // 2, 2, _B))
    return out.reshape(_L, _D, _B).transpose(2, 0, 1)
